# baseline (speedup 1.0000x reference)
import functools

import jax
import jax.numpy as jnp
from jax import lax
from jax.experimental import pallas as pl
from jax.experimental.pallas import tpu as pltpu
from jax.experimental.pallas import tpu_sc as plsc

NW = 32
LANES = 16
JCH = 8
VB = 1024


def _wprep(wt_ref, out_ref):
    t = wt_ref[...].T.reshape(VB // 4, 4, 32)
    for q in range(4):
        out_ref[:, 32 * q:32 * (q + 1)] = t[:, q, :]


def kernel(x, W_E):
    B0, B1 = x.shape
    V, D = W_E.shape
    n_ch = B1 // JCH
    CH = JCH * 128

    n_vb = pl.cdiv(V, VB)
    wp = pl.pallas_call(
        _wprep,
        grid=(n_vb,),
        in_specs=[pl.BlockSpec((D, VB), lambda b: (0, b))],
        out_specs=pl.BlockSpec((VB // 4, 128), lambda b: (b, 0)),
        out_shape=jax.ShapeDtypeStruct((n_vb * VB // 4, 128), jnp.float32),
    )(W_E.T)
    wrow = wp.reshape(n_vb * VB // 4 * 128 // D, D)

    mesh = plsc.VectorSubcoreMesh(core_axis_name="c", subcore_axis_name="s")

    @functools.partial(
        pl.kernel,
        mesh=mesh,
        out_type=jax.ShapeDtypeStruct((B1, B0, D), jnp.float32),
        scratch_types=[
            pltpu.VMEM((128, B1), jnp.int32),
            pltpu.VMEM((B1 * 128,), jnp.int32),
            pltpu.VMEM((CH, D), jnp.float32),
            pltpu.VMEM((CH, D), jnp.float32),
            pltpu.SemaphoreType.DMA,
            pltpu.SemaphoreType.DMA,
            pltpu.SemaphoreType.DMA,
            pltpu.SemaphoreType.DMA,
        ],
        compiler_params=pltpu.CompilerParams(
            use_tc_tiling_on_sc=False, needs_layout_passes=False),
    )
    def emb(x_hbm, w_hbm, un_hbm, xv, idx_v, rows0, rows1, g0, g1, o0, o1):
        w = lax.axis_index("s") * 2 + lax.axis_index("c")
        pltpu.sync_copy(x_hbm.at[pl.ds(128 * w, 128)], xv)
        lane = lax.broadcasted_iota(jnp.int32, (LANES,), 0)

        def tbody(j, carry):
            col = jnp.full((LANES,), j, jnp.int32)
            for g in range(128 // LANES):
                v = plsc.load_gather(xv, [lane + LANES * g, col])
                idx_v[pl.ds(j * 128 + LANES * g, LANES)] = v
            return carry

        lax.fori_loop(0, B1, tbody, 0)

        rows = [rows0, rows1]
        gsem = [g0, g1]
        osem = [o0, o1]
        gather = [None, None]
        wback = [[], []]

        gather[0] = pltpu.async_copy(
            w_hbm.at[idx_v.at[pl.ds(0, CH)]], rows[0], gsem[0])
        for c in range(n_ch):
            b = c % 2
            nb = (c + 1) % 2
            if c + 1 < n_ch:
                for h in wback[nb]:
                    h.wait()
                wback[nb] = []
                gather[nb] = pltpu.async_copy(
                    w_hbm.at[idx_v.at[pl.ds((c + 1) * CH, CH)]],
                    rows[nb], gsem[nb])
            gather[b].wait()
            for jj in range(JCH):
                wback[b].append(pltpu.async_copy(
                    rows[b].at[pl.ds(jj * 128, 128)],
                    un_hbm.at[c * JCH + jj, pl.ds(128 * w, 128)],
                    osem[b]))
        for h in wback[0] + wback[1]:
            h.wait()

    un = emb(x, wrow)
    return un.transpose(1, 0, 2)


# speedup vs baseline: 1.2075x; 1.2075x over previous
import functools

import jax
import jax.numpy as jnp
from jax import lax
from jax.experimental import pallas as pl
from jax.experimental.pallas import tpu as pltpu
from jax.experimental.pallas import tpu_sc as plsc

NW = 32
LANES = 16
JCH = 10
NG = 2


def kernel(x, W_E):
    B0, B1 = x.shape
    D = W_E.shape[1]
    JG = B1 // NG
    n_ch = JG // JCH
    CH = JCH * 128

    mesh = plsc.VectorSubcoreMesh(core_axis_name="c", subcore_axis_name="s")

    def make_emb(g):
        @functools.partial(
            pl.kernel,
            mesh=mesh,
            out_type=jax.ShapeDtypeStruct((JG, B0, D), jnp.float32),
            scratch_types=[
                pltpu.VMEM((128, JG), jnp.int32),
                pltpu.VMEM((JG * 128,), jnp.int32),
                pltpu.VMEM((CH, D), jnp.float32),
                pltpu.VMEM((CH, D), jnp.float32),
                pltpu.SemaphoreType.DMA,
                pltpu.SemaphoreType.DMA,
                pltpu.SemaphoreType.DMA,
                pltpu.SemaphoreType.DMA,
            ],
            compiler_params=pltpu.CompilerParams(
                use_tc_tiling_on_sc=False, needs_layout_passes=False),
        )
        def emb(x_hbm, w_hbm, un_hbm, xv, idx_v, rows0, rows1,
                g0, g1, o0, o1):
            w = lax.axis_index("s") * 2 + lax.axis_index("c")
            pltpu.sync_copy(x_hbm.at[pl.ds(128 * w, 128)], xv)
            lane = lax.broadcasted_iota(jnp.int32, (LANES,), 0)

            def tbody(j, carry):
                col = jnp.full((LANES,), j, jnp.int32)
                for q in range(128 // LANES):
                    v = plsc.load_gather(xv, [lane + LANES * q, col])
                    idx_v[pl.ds(j * 128 + LANES * q, LANES)] = v
                return carry

            lax.fori_loop(0, JG, tbody, 0)

            rows = [rows0, rows1]
            gsem = [g0, g1]
            osem = [o0, o1]
            gather = [None, None]
            wback = [[], []]

            gather[0] = pltpu.async_copy(
                w_hbm.at[idx_v.at[pl.ds(0, CH)]], rows[0], gsem[0])
            for c in range(n_ch):
                b = c % 2
                nb = (c + 1) % 2
                if c + 1 < n_ch:
                    for h in wback[nb]:
                        h.wait()
                    wback[nb] = []
                    gather[nb] = pltpu.async_copy(
                        w_hbm.at[idx_v.at[pl.ds((c + 1) * CH, CH)]],
                        rows[nb], gsem[nb])
                gather[b].wait()
                for jj in range(JCH):
                    wback[b].append(pltpu.async_copy(
                        rows[b].at[pl.ds(jj * 128, 128)],
                        un_hbm.at[c * JCH + jj, pl.ds(128 * w, 128)],
                        osem[b]))
            for h in wback[0] + wback[1]:
                h.wait()

        return emb

    parts = []
    for g in range(NG):
        xg = lax.slice_in_dim(x, g * JG, (g + 1) * JG, axis=1)
        un_g = make_emb(g)(xg, W_E)
        parts.append(un_g.transpose(1, 0, 2))
    return jnp.concatenate(parts, axis=1)


# speedup vs baseline: 1.2530x; 1.0377x over previous
import functools

import jax
import jax.numpy as jnp
from jax import lax
from jax.experimental import pallas as pl
from jax.experimental.pallas import tpu as pltpu
from jax.experimental.pallas import tpu_sc as plsc

NW = 32
JCH = 8


def kernel(x, W_E):
    B0, B1 = x.shape
    D = W_E.shape[1]
    IB = B0 // 128
    n_ch = B1 // JCH
    CH = JCH * 128

    xr = x.reshape(IB, 128, B1).transpose(0, 2, 1)

    mesh = plsc.VectorSubcoreMesh(core_axis_name="c", subcore_axis_name="s")

    @functools.partial(
        pl.kernel,
        mesh=mesh,
        out_type=jax.ShapeDtypeStruct((B1, B0, D), jnp.float32),
        scratch_types=[
            pltpu.VMEM((B1 * 128,), jnp.int32),
            pltpu.VMEM((CH, D), jnp.float32),
            pltpu.VMEM((CH, D), jnp.float32),
            pltpu.SemaphoreType.DMA,
            pltpu.SemaphoreType.DMA,
            pltpu.SemaphoreType.DMA,
            pltpu.SemaphoreType.DMA,
        ],
        compiler_params=pltpu.CompilerParams(
            use_tc_tiling_on_sc=False, needs_layout_passes=False),
    )
    def emb(xr_hbm, w_hbm, un_hbm, idx_v, rows0, rows1, g0, g1, o0, o1):
        w = lax.axis_index("s") * 2 + lax.axis_index("c")
        pltpu.sync_copy(xr_hbm.at[w], idx_v)

        rows = [rows0, rows1]
        gsem = [g0, g1]
        osem = [o0, o1]
        gather = [None, None]
        wback = [[], []]

        gather[0] = pltpu.async_copy(
            w_hbm.at[idx_v.at[pl.ds(0, CH)]], rows[0], gsem[0])
        for c in range(n_ch):
            b = c % 2
            nb = (c + 1) % 2
            if c + 1 < n_ch:
                for h in wback[nb]:
                    h.wait()
                wback[nb] = []
                gather[nb] = pltpu.async_copy(
                    w_hbm.at[idx_v.at[pl.ds((c + 1) * CH, CH)]],
                    rows[nb], gsem[nb])
            gather[b].wait()
            for jj in range(JCH):
                wback[b].append(pltpu.async_copy(
                    rows[b].at[pl.ds(jj * 128, 128)],
                    un_hbm.at[c * JCH + jj, pl.ds(128 * w, 128)],
                    osem[b]))
        for h in wback[0] + wback[1]:
            h.wait()

    un = emb(xr.reshape(IB, B1 * 128), W_E)
    return un.transpose(1, 0, 2)
